# baseline (speedup 1.0000x reference)
import functools

import jax
import jax.numpy as jnp
from jax import lax
from jax.experimental import pallas as pl
from jax.experimental.pallas import tpu as pltpu
from jax.experimental.pallas import tpu_sc as plsc


_LANES = 16


def _sc_rowsum_body(n, nw, adj_hbm, out_hbm, buf0, buf1, outv, sem0, sem1):
    wid = lax.axis_index("s") * 2 + lax.axis_index("c")

    ngroups_total = n // 8
    base = ngroups_total // nw
    rem = ngroups_total % nw
    ngroups = base + jnp.where(wid < rem, 1, 0)
    start_group = base * wid + jnp.minimum(wid, rem)
    start_row = start_group * 8

    row_words = n
    blk_words = 4 * row_words

    def dma(block_idx, buf, sem):
        src = adj_hbm.at[pl.ds((start_row + 4 * block_idx) * row_words,
                               blk_words)]
        return pltpu.make_async_copy(src, buf, sem)

    dma(0, buf0, sem0).start()
    dma(1, buf1, sem1).start()

    nvec = row_words // _LANES

    def reduce_block(buf, lrow_base):
        def body(j, accs):
            col = j * _LANES
            return tuple(accs[r] + buf[pl.ds(r * row_words + col, _LANES)]
                         for r in range(4))
        accs = lax.fori_loop(
            0, nvec, body,
            tuple(jnp.zeros((_LANES,), jnp.float32) for _ in range(4)))
        for r in range(4):
            outv[pl.ds((lrow_base + r) * _LANES, _LANES)] = accs[r]

    def group_body(gi, carry):
        dma(2 * gi, buf0, sem0).wait()
        reduce_block(buf0, 8 * gi)

        @pl.when(gi + 1 < ngroups)
        def _():
            dma(2 * gi + 2, buf0, sem0).start()

        dma(2 * gi + 1, buf1, sem1).wait()
        reduce_block(buf1, 8 * gi + 4)

        @pl.when(gi + 1 < ngroups)
        def _():
            dma(2 * gi + 3, buf1, sem1).start()

        return carry

    lax.fori_loop(0, ngroups, group_body, 0)

    @pl.when(wid < rem)
    def _():
        pltpu.sync_copy(
            outv.at[pl.ds(0, (base + 1) * 8 * _LANES)],
            out_hbm.at[pl.ds(start_row * _LANES, (base + 1) * 8 * _LANES)])

    @pl.when(wid >= rem)
    def _():
        pltpu.sync_copy(
            outv.at[pl.ds(0, base * 8 * _LANES)],
            out_hbm.at[pl.ds(start_row * _LANES, base * 8 * _LANES)])


def _sc_rowsum(adj_flat, n):
    info = plsc.get_sparse_core_info()
    nw = info.num_cores * info.num_subcores
    max_rows = ((n // 8) // nw + 1) * 8
    mesh = plsc.VectorSubcoreMesh(core_axis_name="c", subcore_axis_name="s")
    k = functools.partial(_sc_rowsum_body, n, nw)
    return pl.kernel(
        k,
        mesh=mesh,
        out_type=jax.ShapeDtypeStruct((n * _LANES,), jnp.float32),
        scratch_types=[
            pltpu.VMEM((4 * n,), jnp.float32),
            pltpu.VMEM((4 * n,), jnp.float32),
            pltpu.VMEM((max_rows * _LANES,), jnp.float32),
            pltpu.SemaphoreType.DMA,
            pltpu.SemaphoreType.DMA,
        ],
    )(adj_flat)


def _p1_kernel(x_ref, w1_ref, out_ref):
    out_ref[...] = jnp.dot(x_ref[...], w1_ref[...],
                           preferred_element_type=jnp.float32)


def _pass1_kernel(adj_ref, p1_ref, w2_ref, b1_ref, p2_ref, adjbf_ref):
    a = adj_ref[...]
    h = jnp.dot(a, p1_ref[...], preferred_element_type=jnp.float32)
    h = jnp.maximum(h + b1_ref[...], 0.0)
    p2_ref[...] = jnp.dot(h, w2_ref[...], preferred_element_type=jnp.float32)
    adjbf_ref[...] = a.astype(jnp.bfloat16)


def _pass2_kernel(adjbf_ref, p2_ref, w3_ref, b2_ref, p3_ref):
    a = adjbf_ref[...]
    p2 = p2_ref[...].astype(jnp.bfloat16)
    h = jnp.dot(a, p2, preferred_element_type=jnp.float32)
    h = jnp.maximum(h + b2_ref[...], 0.0)
    p3_ref[...] = jnp.dot(h, w3_ref[...], preferred_element_type=jnp.float32)


def _pass3_kernel(adjbf_ref, p3_ref, x_ref, b3_ref, d_ref, out_ref):
    a = adjbf_ref[...]
    p3 = p3_ref[...].astype(jnp.bfloat16)
    h = jnp.dot(a, p3, preferred_element_type=jnp.float32)
    h = h + b3_ref[...]
    deg = jnp.sum(d_ref[...], axis=1, keepdims=True)
    h = jnp.where(deg == 0.0, x_ref[...], h)
    out_ref[...] = jnp.maximum(h, 0.0)


def kernel(x, adj, Adj, W1, b1, W2, b2, W3, b3):
    n, nfeat = x.shape
    nmid1 = W1.shape[1]
    nmid2 = W2.shape[1]
    nhid = W3.shape[1]

    tm1 = 200 if n % 200 == 0 else n
    tm23 = 400 if n % 400 == 0 else n

    d_partial = _sc_rowsum(Adj.reshape(-1), n).reshape(n, _LANES)

    p1 = pl.pallas_call(
        _p1_kernel,
        out_shape=jax.ShapeDtypeStruct((n, nmid1), jnp.float32),
    )(x, W1)

    p2, adj_bf = pl.pallas_call(
        _pass1_kernel,
        grid=(n // tm1,),
        in_specs=[
            pl.BlockSpec((tm1, n), lambda i: (i, 0)),
            pl.BlockSpec((n, nmid1), lambda i: (0, 0)),
            pl.BlockSpec((nmid1, nmid2), lambda i: (0, 0)),
            pl.BlockSpec((1, nmid1), lambda i: (0, 0)),
        ],
        out_specs=[
            pl.BlockSpec((tm1, nmid2), lambda i: (i, 0)),
            pl.BlockSpec((tm1, n), lambda i: (i, 0)),
        ],
        out_shape=[
            jax.ShapeDtypeStruct((n, nmid2), jnp.float32),
            jax.ShapeDtypeStruct((n, n), jnp.bfloat16),
        ],
        compiler_params=pltpu.CompilerParams(
            dimension_semantics=("arbitrary",)),
    )(adj, p1, W2, b1.reshape(1, -1))

    p3 = pl.pallas_call(
        _pass2_kernel,
        grid=(n // tm23,),
        in_specs=[
            pl.BlockSpec((tm23, n), lambda i: (i, 0)),
            pl.BlockSpec((n, nmid2), lambda i: (0, 0)),
            pl.BlockSpec((nmid2, nhid), lambda i: (0, 0)),
            pl.BlockSpec((1, nmid2), lambda i: (0, 0)),
        ],
        out_specs=pl.BlockSpec((tm23, nhid), lambda i: (i, 0)),
        out_shape=jax.ShapeDtypeStruct((n, nhid), jnp.float32),
        compiler_params=pltpu.CompilerParams(
            dimension_semantics=("arbitrary",)),
    )(adj_bf, p2, W3, b2.reshape(1, -1))

    out = pl.pallas_call(
        _pass3_kernel,
        grid=(n // tm23,),
        in_specs=[
            pl.BlockSpec((tm23, n), lambda i: (i, 0)),
            pl.BlockSpec((n, nhid), lambda i: (0, 0)),
            pl.BlockSpec((tm23, nfeat), lambda i: (i, 0)),
            pl.BlockSpec((1, nhid), lambda i: (0, 0)),
            pl.BlockSpec((tm23, _LANES), lambda i: (i, 0)),
        ],
        out_specs=pl.BlockSpec((tm23, nhid), lambda i: (i, 0)),
        out_shape=jax.ShapeDtypeStruct((n, nhid), jnp.float32),
        compiler_params=pltpu.CompilerParams(
            dimension_semantics=("arbitrary",)),
    )(adj_bf, p3, x, b3.reshape(1, -1), d_partial)

    return out


# speedup vs baseline: 2.1369x; 2.1369x over previous
import jax
import jax.numpy as jnp
from jax.experimental import pallas as pl
from jax.experimental.pallas import tpu as pltpu

_F8 = jnp.float8_e4m3fn


def _p1_kernel(x_ref, w1_ref, out_ref):
    out_ref[...] = jnp.dot(x_ref[...], w1_ref[...],
                           preferred_element_type=jnp.float32)


def _pass1_kernel(adj_ref, big_ref, p1_ref, w2_ref, b1_ref,
                  p2_ref, adj8_ref, d_ref):
    a = adj_ref[...]
    h = jnp.dot(a, p1_ref[...], preferred_element_type=jnp.float32)
    h = jnp.maximum(h + b1_ref[...], 0.0)
    p2_ref[...] = jnp.dot(h, w2_ref[...], preferred_element_type=jnp.float32)
    adj8_ref[...] = a.astype(_F8)
    d_ref[...] = jnp.sum(big_ref[...], axis=1, keepdims=True)


def _pass2_kernel(adj8_ref, p2_ref, w3_ref, b2_ref, p3_ref):
    a = adj8_ref[...].astype(jnp.bfloat16)
    p2 = p2_ref[...].astype(jnp.bfloat16)
    h = jnp.dot(a, p2, preferred_element_type=jnp.float32)
    h = jnp.maximum(h + b2_ref[...], 0.0)
    p3_ref[...] = jnp.dot(h, w3_ref[...], preferred_element_type=jnp.float32)


def _pass3_kernel(adj8_ref, p3_ref, x_ref, b3_ref, d_ref, out_ref):
    a = adj8_ref[...].astype(jnp.bfloat16)
    p3 = p3_ref[...].astype(jnp.bfloat16)
    h = jnp.dot(a, p3, preferred_element_type=jnp.float32)
    h = h + b3_ref[...]
    h = jnp.where(d_ref[...] == 0.0, x_ref[...], h)
    out_ref[...] = jnp.maximum(h, 0.0)


def kernel(x, adj, Adj, W1, b1, W2, b2, W3, b3):
    n, nfeat = x.shape
    nmid1 = W1.shape[1]
    nmid2 = W2.shape[1]
    nhid = W3.shape[1]

    tm1 = 200 if n % 200 == 0 else n
    tm23 = 400 if n % 400 == 0 else n

    p1 = pl.pallas_call(
        _p1_kernel,
        out_shape=jax.ShapeDtypeStruct((n, nmid1), jnp.float32),
    )(x, W1)

    p2, adj8, d = pl.pallas_call(
        _pass1_kernel,
        grid=(n // tm1,),
        in_specs=[
            pl.BlockSpec((tm1, n), lambda i: (i, 0)),
            pl.BlockSpec((tm1, n), lambda i: (i, 0)),
            pl.BlockSpec((n, nmid1), lambda i: (0, 0)),
            pl.BlockSpec((nmid1, nmid2), lambda i: (0, 0)),
            pl.BlockSpec((1, nmid1), lambda i: (0, 0)),
        ],
        out_specs=[
            pl.BlockSpec((tm1, nmid2), lambda i: (i, 0)),
            pl.BlockSpec((tm1, n), lambda i: (i, 0)),
            pl.BlockSpec((tm1, 1), lambda i: (i, 0)),
        ],
        out_shape=[
            jax.ShapeDtypeStruct((n, nmid2), jnp.float32),
            jax.ShapeDtypeStruct((n, n), _F8),
            jax.ShapeDtypeStruct((n, 1), jnp.float32),
        ],
        compiler_params=pltpu.CompilerParams(
            dimension_semantics=("arbitrary",)),
    )(adj, Adj, p1, W2, b1.reshape(1, -1))

    p3 = pl.pallas_call(
        _pass2_kernel,
        grid=(n // tm23,),
        in_specs=[
            pl.BlockSpec((tm23, n), lambda i: (i, 0)),
            pl.BlockSpec((n, nmid2), lambda i: (0, 0)),
            pl.BlockSpec((nmid2, nhid), lambda i: (0, 0)),
            pl.BlockSpec((1, nmid2), lambda i: (0, 0)),
        ],
        out_specs=pl.BlockSpec((tm23, nhid), lambda i: (i, 0)),
        out_shape=jax.ShapeDtypeStruct((n, nhid), jnp.float32),
        compiler_params=pltpu.CompilerParams(
            dimension_semantics=("arbitrary",)),
    )(adj8, p2, W3, b2.reshape(1, -1))

    out = pl.pallas_call(
        _pass3_kernel,
        grid=(n // tm23,),
        in_specs=[
            pl.BlockSpec((tm23, n), lambda i: (i, 0)),
            pl.BlockSpec((n, nhid), lambda i: (0, 0)),
            pl.BlockSpec((tm23, nfeat), lambda i: (i, 0)),
            pl.BlockSpec((1, nhid), lambda i: (0, 0)),
            pl.BlockSpec((tm23, 1), lambda i: (i, 0)),
        ],
        out_specs=pl.BlockSpec((tm23, nhid), lambda i: (i, 0)),
        out_shape=jax.ShapeDtypeStruct((n, nhid), jnp.float32),
        compiler_params=pltpu.CompilerParams(
            dimension_semantics=("arbitrary",)),
    )(adj8, p3, x, b3.reshape(1, -1), d)

    return out


# speedup vs baseline: 2.3092x; 1.0806x over previous
import jax
import jax.numpy as jnp
from jax.experimental import pallas as pl
from jax.experimental.pallas import tpu as pltpu

_F8 = jnp.float8_e4m3fn


def _p1_kernel(x_ref, w1_ref, out_ref):
    out_ref[...] = jnp.dot(x_ref[...], w1_ref[...],
                           preferred_element_type=jnp.float32)


_PSCALE = 0.0078125


def _pass1_kernel(adj_ref, big_ref, p1_ref, w2_ref, b1_ref,
                  p2_ref, adj8_ref, d_ref):
    a = adj_ref[...]
    h = jnp.dot(a, p1_ref[...], preferred_element_type=jnp.float32)
    h = jnp.maximum(h + b1_ref[...], 0.0)
    p2 = jnp.dot(h, w2_ref[...], preferred_element_type=jnp.float32)
    p2_ref[...] = (p2 * _PSCALE).astype(_F8)
    adj8_ref[...] = a.astype(_F8)
    d_ref[...] = jnp.sum(big_ref[...], axis=1, keepdims=True)


def _pass2_kernel(adj8_ref, p2_ref, w3_ref, b2_ref, p3_ref):
    a = adj8_ref[...]
    h = jnp.dot(a, p2_ref[...], preferred_element_type=jnp.float32)
    h = jnp.maximum(h + b2_ref[...], 0.0)
    p3 = jnp.dot(h, w3_ref[...], preferred_element_type=jnp.float32)
    p3_ref[...] = p3.astype(_F8)


def _pass3_kernel(adj8_ref, p3_ref, x_ref, b3_ref, d_ref, out_ref):
    a = adj8_ref[...]
    h = jnp.dot(a, p3_ref[...], preferred_element_type=jnp.float32)
    h = h * 128.0 + b3_ref[...]
    h = jnp.where(d_ref[...] == 0.0, x_ref[...], h)
    out_ref[...] = jnp.maximum(h, 0.0)


def kernel(x, adj, Adj, W1, b1, W2, b2, W3, b3):
    n, nfeat = x.shape
    nmid1 = W1.shape[1]
    nmid2 = W2.shape[1]
    nhid = W3.shape[1]

    tm1 = 200 if n % 200 == 0 else n
    tm23 = 400 if n % 400 == 0 else n

    p1 = pl.pallas_call(
        _p1_kernel,
        out_shape=jax.ShapeDtypeStruct((n, nmid1), jnp.float32),
    )(x, W1)

    p2, adj8, d = pl.pallas_call(
        _pass1_kernel,
        grid=(n // tm1,),
        in_specs=[
            pl.BlockSpec((tm1, n), lambda i: (i, 0)),
            pl.BlockSpec((tm1, n), lambda i: (i, 0)),
            pl.BlockSpec((n, nmid1), lambda i: (0, 0)),
            pl.BlockSpec((nmid1, nmid2), lambda i: (0, 0)),
            pl.BlockSpec((1, nmid1), lambda i: (0, 0)),
        ],
        out_specs=[
            pl.BlockSpec((tm1, nmid2), lambda i: (i, 0)),
            pl.BlockSpec((tm1, n), lambda i: (i, 0)),
            pl.BlockSpec((tm1, 1), lambda i: (i, 0)),
        ],
        out_shape=[
            jax.ShapeDtypeStruct((n, nmid2), _F8),
            jax.ShapeDtypeStruct((n, n), _F8),
            jax.ShapeDtypeStruct((n, 1), jnp.float32),
        ],
        compiler_params=pltpu.CompilerParams(
            dimension_semantics=("arbitrary",)),
    )(adj, Adj, p1, W2, b1.reshape(1, -1))

    p3 = pl.pallas_call(
        _pass2_kernel,
        grid=(n // tm23,),
        in_specs=[
            pl.BlockSpec((tm23, n), lambda i: (i, 0)),
            pl.BlockSpec((n, nmid2), lambda i: (0, 0)),
            pl.BlockSpec((nmid2, nhid), lambda i: (0, 0)),
            pl.BlockSpec((1, nmid2), lambda i: (0, 0)),
        ],
        out_specs=pl.BlockSpec((tm23, nhid), lambda i: (i, 0)),
        out_shape=jax.ShapeDtypeStruct((n, nhid), _F8),
        compiler_params=pltpu.CompilerParams(
            dimension_semantics=("arbitrary",)),
    )(adj8, p2, W3, (b2 * _PSCALE).reshape(1, -1))

    out = pl.pallas_call(
        _pass3_kernel,
        grid=(n // tm23,),
        in_specs=[
            pl.BlockSpec((tm23, n), lambda i: (i, 0)),
            pl.BlockSpec((n, nhid), lambda i: (0, 0)),
            pl.BlockSpec((tm23, nfeat), lambda i: (i, 0)),
            pl.BlockSpec((1, nhid), lambda i: (0, 0)),
            pl.BlockSpec((tm23, 1), lambda i: (i, 0)),
        ],
        out_specs=pl.BlockSpec((tm23, nhid), lambda i: (i, 0)),
        out_shape=jax.ShapeDtypeStruct((n, nhid), jnp.float32),
        compiler_params=pltpu.CompilerParams(
            dimension_semantics=("arbitrary",)),
    )(adj8, p3, x, b3.reshape(1, -1), d)

    return out
